# Initial kernel scaffold; baseline (speedup 1.0000x reference)
import functools

import jax
import jax.numpy as jnp
from jax import lax
from jax.experimental import pallas as pl
from jax.experimental.pallas import tpu as pltpu
from jax.experimental.pallas import tpu_sc as plsc

N = 100000
E = 1600000
G = 64
NPAD = 102400
NC, NS = 2, 16
ROWS_PER_TILE = NPAD // NS
CK = 2000
TB = 12800
NB = NPAD // TB

_MESH = dict(core_axis_name="c", subcore_axis_name="s",
             num_cores=NC, num_subcores=NS)


def _sc_deg(dst, ones_ck, zrows):
    ept = E // (NC * NS)
    nch = ept // CK

    @functools.partial(
        pl.kernel,
        out_type=jax.ShapeDtypeStruct((NC, NPAD), jnp.float32),
        mesh=plsc.VectorSubcoreMesh(**_MESH),
        scratch_types=[
            pltpu.VMEM((CK,), jnp.int32),
            pltpu.VMEM((CK,), jnp.float32),
            pltpu.VMEM_SHARED((NPAD,), jnp.float32),
        ],
    )
    def k(dst_hbm, ones_hbm, z_hbm, out_hbm, idx_v, ones_v, acc_sh):
        c = lax.axis_index("c")
        s = lax.axis_index("s")
        r0 = s * ROWS_PER_TILE
        pltpu.sync_copy(ones_hbm, ones_v)
        pltpu.sync_copy(z_hbm.at[pl.ds(r0, ROWS_PER_TILE)],
                        acc_sh.at[pl.ds(r0, ROWS_PER_TILE)])
        plsc.subcore_barrier()

        def body(j, carry):
            base = (c * NS + s) * ept + j * CK
            pltpu.sync_copy(dst_hbm.at[pl.ds(base, CK)], idx_v)
            pltpu.sync_copy(ones_v, acc_sh.at[idx_v], add=True)
            return carry

        lax.fori_loop(0, nch, body, 0)
        plsc.subcore_barrier()
        pltpu.sync_copy(acc_sh.at[pl.ds(r0, ROWS_PER_TILE)],
                        out_hbm.at[c, pl.ds(r0, ROWS_PER_TILE)])

    return k(dst, ones_ck, zrows)


def _sc_conv(y, src_idx, dst, zrows, F, split_edges):
    if split_edges:
        ept = E // (NC * NS)
        src_stride, dst_stride = E // NC, E // NC
    else:
        ept = E // NS
        src_stride, dst_stride = E, 0
    nch = ept // CK

    @functools.partial(
        pl.kernel,
        out_type=jax.ShapeDtypeStruct((NC, NPAD, F), jnp.float32),
        mesh=plsc.VectorSubcoreMesh(**_MESH),
        scratch_types=[
            pltpu.VMEM((CK,), jnp.int32),
            pltpu.VMEM((CK,), jnp.int32),
            pltpu.VMEM((CK, F), jnp.float32),
            pltpu.VMEM_SHARED((NPAD, F), jnp.float32),
        ],
    )
    def k(y_hbm, src_hbm, dst_hbm, z_hbm, out_hbm,
          src_v, dst_v, rows_v, acc_sh):
        c = lax.axis_index("c")
        s = lax.axis_index("s")
        r0 = s * ROWS_PER_TILE
        pltpu.sync_copy(z_hbm.at[pl.ds(r0, ROWS_PER_TILE)],
                        acc_sh.at[pl.ds(r0, ROWS_PER_TILE)])
        plsc.subcore_barrier()

        def body(j, carry):
            ebase = s * ept + j * CK
            pltpu.sync_copy(src_hbm.at[pl.ds(c * src_stride + ebase, CK)],
                            src_v)
            pltpu.sync_copy(dst_hbm.at[pl.ds(c * dst_stride + ebase, CK)],
                            dst_v)
            pltpu.sync_copy(y_hbm.at[src_v], rows_v)
            pltpu.sync_copy(rows_v, acc_sh.at[dst_v], add=True)
            return carry

        lax.fori_loop(0, nch, body, 0)
        plsc.subcore_barrier()
        pltpu.sync_copy(acc_sh.at[pl.ds(r0, ROWS_PER_TILE)],
                        out_hbm.at[c, pl.ds(r0, ROWS_PER_TILE)])

    return k(y, src_idx, dst, zrows)


def _tc_head(xp, degp, W1, b1, Wc1):
    def body(x_ref, degp_ref, W1_ref, b1_ref, Wc1_ref, y_ref, dinv_ref):
        deg = degp_ref[0, :] + degp_ref[1, :] + 1.0
        dinv = lax.rsqrt(deg)
        h = jnp.maximum(
            jnp.dot(x_ref[...], W1_ref[...],
                    preferred_element_type=jnp.float32) + b1_ref[...][None, :],
            0.0)
        xw = jnp.dot(h, Wc1_ref[...], preferred_element_type=jnp.float32)
        y = xw * dinv[:, None]
        y_ref[0, :, :] = y[:, :16]
        y_ref[1, :, :] = y[:, 16:]
        dinv_ref[...] = dinv

    return pl.pallas_call(
        body,
        grid=(NB,),
        in_specs=[
            pl.BlockSpec((TB, 12), lambda i: (i, 0)),
            pl.BlockSpec((2, TB), lambda i: (0, i)),
            pl.BlockSpec((12, 32), lambda i: (0, 0)),
            pl.BlockSpec((32,), lambda i: (0,)),
            pl.BlockSpec((32, 32), lambda i: (0, 0)),
        ],
        out_specs=[
            pl.BlockSpec((2, TB, 16), lambda i: (0, i, 0)),
            pl.BlockSpec((TB,), lambda i: (i,)),
        ],
        out_shape=[
            jax.ShapeDtypeStruct((2, NPAD, 16), jnp.float32),
            jax.ShapeDtypeStruct((NPAD,), jnp.float32),
        ],
    )(xp, degp, W1, b1, Wc1)


def _tc_mid(agg1, y1, dinv, bc1, Wc3p):
    def body(agg_ref, y1_ref, dinv_ref, bc1_ref, Wc3_ref, y2_ref):
        dinv = dinv_ref[...]
        lo = jnp.maximum(
            dinv[:, None] * (agg_ref[0] + y1_ref[0]) + bc1_ref[0:16][None, :],
            0.0)
        hi = jnp.maximum(
            dinv[:, None] * (agg_ref[1] + y1_ref[1]) + bc1_ref[16:32][None, :],
            0.0)
        xw = (jnp.dot(lo, Wc3_ref[0:16, :], preferred_element_type=jnp.float32)
              + jnp.dot(hi, Wc3_ref[16:32, :],
                        preferred_element_type=jnp.float32))
        y2_ref[...] = xw * dinv[:, None]

    return pl.pallas_call(
        body,
        grid=(NB,),
        in_specs=[
            pl.BlockSpec((2, TB, 16), lambda i: (0, i, 0)),
            pl.BlockSpec((2, TB, 16), lambda i: (0, i, 0)),
            pl.BlockSpec((TB,), lambda i: (i,)),
            pl.BlockSpec((32,), lambda i: (0,)),
            pl.BlockSpec((32, 4), lambda i: (0, 0)),
        ],
        out_specs=pl.BlockSpec((TB, 4), lambda i: (i, 0)),
        out_shape=jax.ShapeDtypeStruct((NPAD, 4), jnp.float32),
    )(agg1, y1, dinv, bc1, Wc3p)


def _tc_tail(agg2, y2, dinv, bc3p, batchp):
    def body(agg_ref, y2_ref, dinv_ref, bc3_ref, batch_ref, out_ref):
        i = pl.program_id(0)
        dinv = dinv_ref[...]
        h3 = (dinv[:, None] * (agg_ref[0] + agg_ref[1] + y2_ref[...])
              + bc3_ref[...][None, :])
        oh = (batch_ref[...][:, None]
              == lax.broadcasted_iota(jnp.int32, (1, G), 1)).astype(jnp.float32)
        blk = lax.dot_general(oh, h3, (((0,), (0,)), ((), ())),
                              preferred_element_type=jnp.float32)

        @pl.when(i == 0)
        def _():
            out_ref[...] = jnp.zeros((G, 4), jnp.float32)

        out_ref[...] += blk

        @pl.when(i == NB - 1)
        def _():
            p = out_ref[...]
            col = lax.broadcasted_iota(jnp.int32, (G, 4), 1)
            pm = jnp.where(col < 3, p, -jnp.inf)
            mx = jnp.max(pm, axis=1, keepdims=True)
            lse = mx + jnp.log(
                jnp.sum(jnp.where(col < 3, jnp.exp(p - mx), 0.0),
                        axis=1, keepdims=True))
            out_ref[...] = p - lse

    return pl.pallas_call(
        body,
        grid=(NB,),
        in_specs=[
            pl.BlockSpec((2, TB, 4), lambda i: (0, i, 0)),
            pl.BlockSpec((TB, 4), lambda i: (i, 0)),
            pl.BlockSpec((TB,), lambda i: (i,)),
            pl.BlockSpec((4,), lambda i: (0,)),
            pl.BlockSpec((TB,), lambda i: (i,)),
        ],
        out_specs=pl.BlockSpec((G, 4), lambda i: (0, 0)),
        out_shape=jax.ShapeDtypeStruct((G, 4), jnp.float32),
    )(agg2, y2, dinv, bc3p, batchp)


def kernel(x, edge_index, batch, W1, b1, Wc1, bc1, Wc3, bc3):
    src = edge_index[0]
    dst = edge_index[1]
    src2 = jnp.concatenate([src, src + NPAD])
    xp = jnp.pad(x, ((0, NPAD - N), (0, 0)))
    batchp = jnp.pad(batch, (0, NPAD - N), constant_values=G)
    Wc3p = jnp.pad(Wc3, ((0, 0), (0, 1)))
    bc3p = jnp.pad(bc3, (0, 1))
    ones_ck = jnp.ones((CK,), jnp.float32)
    z1 = jnp.zeros((NPAD,), jnp.float32)
    z16 = jnp.zeros((NPAD, 16), jnp.float32)
    z4 = jnp.zeros((NPAD, 4), jnp.float32)

    degp = _sc_deg(dst, ones_ck, z1)
    y1, dinv = _tc_head(xp, degp, W1, b1, Wc1)
    agg1 = _sc_conv(y1.reshape(2 * NPAD, 16), src2, dst, z16,
                    F=16, split_edges=False)
    y2 = _tc_mid(agg1, y1, dinv, bc1, Wc3p)
    agg2 = _sc_conv(y2, src, dst, z4, F=4, split_edges=True)
    pooled = _tc_tail(agg2, y2, dinv, bc3p, batchp)
    return pooled[:, :3]


# speedup vs baseline: 48.2064x; 48.2064x over previous
import functools

import jax
import jax.numpy as jnp
from jax import lax
from jax.experimental import pallas as pl
from jax.experimental.pallas import tpu as pltpu
from jax.experimental.pallas import tpu_sc as plsc

N = 100000
E = 1600000
G = 64
NPAD = 100352
NC, NS = 2, 16
RPT = NPAD // NS
TB = 12544
NB = NPAD // TB

_MESH = dict(core_axis_name="c", subcore_axis_name="s",
             num_cores=NC, num_subcores=NS)
_SC_PARAMS = pltpu.CompilerParams(use_tc_tiling_on_sc=False)
_TC_PARAMS = pltpu.CompilerParams(vmem_limit_bytes=110 * 1024 * 1024)


def _sc_deg(dst, zrows):
    ck = 2000
    ept = E // (NC * NS)
    nch = ept // ck

    @functools.partial(
        pl.kernel,
        out_type=jax.ShapeDtypeStruct((NC, NPAD), jnp.float32),
        mesh=plsc.VectorSubcoreMesh(**_MESH),
        compiler_params=_SC_PARAMS,
        scratch_types=[
            pltpu.VMEM((ck,), jnp.int32),
            pltpu.VMEM((ck,), jnp.int32),
            pltpu.VMEM((ck,), jnp.float32),
            pltpu.VMEM_SHARED((NPAD,), jnp.float32),
            pltpu.SemaphoreType.DMA,
            pltpu.SemaphoreType.DMA,
        ],
    )
    def k(dst_hbm, z_hbm, out_hbm, dst0_v, dst1_v, ones_v, acc_sh, s0, s1):
        c = lax.axis_index("c")
        s = lax.axis_index("s")
        r0 = s * RPT
        dbuf = (dst0_v, dst1_v)
        sem = (s0, s1)

        def fill(i, carry):
            ones_v[pl.ds(i * 16, 16)] = jnp.ones((16,), jnp.float32)
            return carry

        lax.fori_loop(0, ck // 16, fill, 0)
        pltpu.sync_copy(z_hbm.at[pl.ds(r0, RPT)], acc_sh.at[pl.ds(r0, RPT)])
        plsc.subcore_barrier()

        def idx(j, b):
            base = (c * NS + s) * ept + j * ck
            pltpu.sync_copy(dst_hbm.at[pl.ds(base, ck)], dbuf[b])

        def sstart(b):
            pltpu.async_copy(ones_v, acc_sh.at[dbuf[b]], sem[b], add=True)

        def swait(b):
            pltpu.make_async_copy(ones_v, acc_sh.at[dbuf[b]], sem[b]).wait()

        idx(0, 0)

        def body(j2, carry):
            e = 2 * j2
            idx(e + 1, 1)
            sstart(0)
            swait(0)

            @pl.when(e + 2 < nch)
            def _():
                idx(e + 2, 0)

            sstart(1)
            swait(1)
            return carry

        lax.fori_loop(0, nch // 2, body, 0)
        if nch % 2 == 1:
            sstart(0)
            swait(0)
        plsc.subcore_barrier()
        pltpu.sync_copy(acc_sh.at[pl.ds(r0, RPT)],
                        out_hbm.at[c, pl.ds(r0, RPT)])

    return k(dst, zrows)


def _sc_conv(y, src, dst, zrows, F, split_edges, ck):
    if split_edges:
        ept = E // (NC * NS)
        estride = E // NC
        off_core = False
    else:
        ept = E // NS
        estride = 0
        off_core = True
    nch = ept // ck

    @functools.partial(
        pl.kernel,
        out_type=jax.ShapeDtypeStruct((NC, NPAD, F), jnp.float32),
        mesh=plsc.VectorSubcoreMesh(**_MESH),
        compiler_params=_SC_PARAMS,
        scratch_types=[
            pltpu.VMEM((ck,), jnp.int32),
            pltpu.VMEM((ck,), jnp.int32),
            pltpu.VMEM((ck,), jnp.int32),
            pltpu.VMEM((ck,), jnp.int32),
            pltpu.VMEM((ck, F), jnp.float32),
            pltpu.VMEM((ck, F), jnp.float32),
            pltpu.VMEM_SHARED((NPAD, F), jnp.float32),
            pltpu.SemaphoreType.DMA,
            pltpu.SemaphoreType.DMA,
            pltpu.SemaphoreType.DMA,
            pltpu.SemaphoreType.DMA,
        ],
    )
    def k(y_hbm, src_hbm, dst_hbm, z_hbm, out_hbm,
          src0_v, src1_v, dst0_v, dst1_v, rows0_v, rows1_v, acc_sh,
          g0, g1, s0, s1):
        c = lax.axis_index("c")
        s = lax.axis_index("s")
        r0 = s * RPT
        sbuf = (src0_v, src1_v)
        dbuf = (dst0_v, dst1_v)
        rbuf = (rows0_v, rows1_v)
        gsem = (g0, g1)
        ssem = (s0, s1)

        pltpu.sync_copy(z_hbm.at[pl.ds(r0, RPT)], acc_sh.at[pl.ds(r0, RPT)])
        plsc.subcore_barrier()

        def idx(j, b):
            base = s * ept + j * ck
            pltpu.sync_copy(src_hbm.at[pl.ds(c * estride + base, ck)],
                            sbuf[b])
            pltpu.sync_copy(dst_hbm.at[pl.ds(c * estride + base, ck)],
                            dbuf[b])
            if off_core:
                off = jnp.full((16,), c * NPAD, jnp.int32)

                def addoff(i, carry):
                    sbuf[b][pl.ds(i * 16, 16)] = (
                        sbuf[b][pl.ds(i * 16, 16)] + off)
                    return carry

                lax.fori_loop(0, ck // 16, addoff, 0)

        def gstart(b):
            pltpu.async_copy(y_hbm.at[sbuf[b]], rbuf[b], gsem[b])

        def gwait(b):
            pltpu.make_async_copy(y_hbm.at[sbuf[b]], rbuf[b],
                                  gsem[b]).wait()

        def sstart(b):
            pltpu.async_copy(rbuf[b], acc_sh.at[dbuf[b]], ssem[b], add=True)

        def swait(b):
            pltpu.make_async_copy(rbuf[b], acc_sh.at[dbuf[b]],
                                  ssem[b]).wait()

        idx(0, 0)
        gstart(0)

        def body(j2, carry):
            e = 2 * j2
            idx(e + 1, 1)
            gstart(1)
            gwait(0)
            sstart(0)
            swait(0)

            @pl.when(e + 2 < nch)
            def _():
                idx(e + 2, 0)
                gstart(0)

            gwait(1)
            sstart(1)
            swait(1)
            return carry

        lax.fori_loop(0, nch // 2, body, 0)
        if nch % 2 == 1:
            gwait(0)
            sstart(0)
            swait(0)
        plsc.subcore_barrier()
        pltpu.sync_copy(acc_sh.at[pl.ds(r0, RPT)],
                        out_hbm.at[c, pl.ds(r0, RPT)])

    return k(y, src, dst, zrows)


def _tc_head(xp, degp, W1, b1, Wc1):
    def body(x_ref, degp_ref, W1_ref, b1_ref, Wc1_ref, y_ref, dinv_ref):
        deg = degp_ref[0, :] + degp_ref[1, :] + 1.0
        dinv = lax.rsqrt(deg)
        h = jnp.maximum(
            jnp.dot(x_ref[...], W1_ref[...],
                    preferred_element_type=jnp.float32) + b1_ref[...][None, :],
            0.0)
        xw = jnp.dot(h, Wc1_ref[...], preferred_element_type=jnp.float32)
        y = xw * dinv[:, None]
        y_ref[0, :, :] = y[:, :16]
        y_ref[1, :, :] = y[:, 16:]
        dinv_ref[0, 0, :] = dinv

    return pl.pallas_call(
        body,
        grid=(NB,),
        in_specs=[
            pl.BlockSpec((TB, 12), lambda i: (i, 0)),
            pl.BlockSpec((2, TB), lambda i: (0, i)),
            pl.BlockSpec((12, 32), lambda i: (0, 0)),
            pl.BlockSpec((32,), lambda i: (0,)),
            pl.BlockSpec((32, 32), lambda i: (0, 0)),
        ],
        out_specs=[
            pl.BlockSpec((2, TB, 16), lambda i: (0, i, 0)),
            pl.BlockSpec((1, 1, TB), lambda i: (i, 0, 0)),
        ],
        out_shape=[
            jax.ShapeDtypeStruct((2, NPAD, 16), jnp.float32),
            jax.ShapeDtypeStruct((NB, 1, TB), jnp.float32),
        ],
        compiler_params=_TC_PARAMS,
    )(xp, degp, W1, b1, Wc1)


def _tc_mid(agg1, y1, dinv, bc1, Wc3p):
    def body(agg_ref, y1_ref, dinv_ref, bc1_ref, Wc3_ref, y2_ref):
        dinv = dinv_ref[0, 0, :]
        lo = jnp.maximum(
            dinv[:, None] * (agg_ref[0] + y1_ref[0]) + bc1_ref[0:16][None, :],
            0.0)
        hi = jnp.maximum(
            dinv[:, None] * (agg_ref[1] + y1_ref[1]) + bc1_ref[16:32][None, :],
            0.0)
        xw = (jnp.dot(lo, Wc3_ref[0:16, :], preferred_element_type=jnp.float32)
              + jnp.dot(hi, Wc3_ref[16:32, :],
                        preferred_element_type=jnp.float32))
        y2_ref[...] = xw * dinv[:, None]

    return pl.pallas_call(
        body,
        grid=(NB,),
        in_specs=[
            pl.BlockSpec((2, TB, 16), lambda i: (0, i, 0)),
            pl.BlockSpec((2, TB, 16), lambda i: (0, i, 0)),
            pl.BlockSpec((1, 1, TB), lambda i: (i, 0, 0)),
            pl.BlockSpec((32,), lambda i: (0,)),
            pl.BlockSpec((32, 8), lambda i: (0, 0)),
        ],
        out_specs=pl.BlockSpec((TB, 8), lambda i: (i, 0)),
        out_shape=jax.ShapeDtypeStruct((NPAD, 8), jnp.float32),
        compiler_params=_TC_PARAMS,
    )(agg1, y1, dinv, bc1, Wc3p)


def _tc_tail(agg2, y2, dinv, bc3p, batchp):
    def body(agg_ref, y2_ref, dinv_ref, bc3_ref, batch_ref, out_ref):
        i = pl.program_id(0)
        dinv = dinv_ref[0, 0, :]
        h3 = (dinv[:, None] * (agg_ref[0] + agg_ref[1] + y2_ref[...])
              + bc3_ref[...][None, :])
        oh = (batch_ref[0, 0, :][:, None]
              == lax.broadcasted_iota(jnp.int32, (1, G), 1)).astype(jnp.float32)
        blk = lax.dot_general(oh, h3, (((0,), (0,)), ((), ())),
                              preferred_element_type=jnp.float32)

        @pl.when(i == 0)
        def _():
            out_ref[...] = jnp.zeros((G, 8), jnp.float32)

        out_ref[...] += blk

        @pl.when(i == NB - 1)
        def _():
            p = out_ref[...]
            col = lax.broadcasted_iota(jnp.int32, (G, 8), 1)
            pm = jnp.where(col < 3, p, -jnp.inf)
            mx = jnp.max(pm, axis=1, keepdims=True)
            lse = mx + jnp.log(
                jnp.sum(jnp.where(col < 3, jnp.exp(p - mx), 0.0),
                        axis=1, keepdims=True))
            out_ref[...] = p - lse

    return pl.pallas_call(
        body,
        grid=(NB,),
        in_specs=[
            pl.BlockSpec((2, TB, 8), lambda i: (0, i, 0)),
            pl.BlockSpec((TB, 8), lambda i: (i, 0)),
            pl.BlockSpec((1, 1, TB), lambda i: (i, 0, 0)),
            pl.BlockSpec((8,), lambda i: (0,)),
            pl.BlockSpec((1, 1, TB), lambda i: (i, 0, 0)),
        ],
        out_specs=pl.BlockSpec((G, 8), lambda i: (0, 0)),
        out_shape=jax.ShapeDtypeStruct((G, 8), jnp.float32),
        compiler_params=_TC_PARAMS,
    )(agg2, y2, dinv, bc3p, batchp)


def kernel(x, edge_index, batch, W1, b1, Wc1, bc1, Wc3, bc3):
    src = edge_index[0]
    dst = edge_index[1]
    xp = jnp.pad(x, ((0, NPAD - N), (0, 0)))
    batchp = jnp.pad(batch, (0, NPAD - N),
                     constant_values=G).reshape(NB, 1, TB)
    Wc3p = jnp.pad(Wc3, ((0, 0), (0, 5)))
    bc3p = jnp.pad(bc3, (0, 5))
    z1 = jnp.zeros((NPAD,), jnp.float32)
    z16 = jnp.zeros((NPAD, 16), jnp.float32)
    z8 = jnp.zeros((NPAD, 8), jnp.float32)

    degp = _sc_deg(dst, z1)
    y1, dinv = _tc_head(xp, degp, W1, b1, Wc1)
    agg1 = _sc_conv(y1.reshape(2 * NPAD, 16), src, dst, z16,
                    F=16, split_edges=False, ck=800)
    y2 = _tc_mid(agg1, y1, dinv, bc1, Wc3p)
    agg2 = _sc_conv(y2, src, dst, z8, F=8, split_edges=True, ck=1000)
    pooled = _tc_tail(agg2, y2, dinv, bc3p, batchp)
    return pooled[:, :3]


# speedup vs baseline: 50.7551x; 1.0529x over previous
import functools

import jax
import jax.numpy as jnp
from jax import lax
from jax.experimental import pallas as pl
from jax.experimental.pallas import tpu as pltpu
from jax.experimental.pallas import tpu_sc as plsc

N = 100000
E = 1600000
G = 64
NPAD = 100352
NC, NS = 2, 16
RPT = NPAD // NS
TB = 12544
NB = NPAD // TB

_MESH = dict(core_axis_name="c", subcore_axis_name="s",
             num_cores=NC, num_subcores=NS)
_SC_PARAMS = pltpu.CompilerParams(use_tc_tiling_on_sc=False)
_TC_PARAMS = pltpu.CompilerParams(vmem_limit_bytes=110 * 1024 * 1024)


def _sc_deg(ei):
    ck = 2000
    ept = E // (NC * NS)
    nch = ept // ck

    @functools.partial(
        pl.kernel,
        out_type=jax.ShapeDtypeStruct((NC, NPAD), jnp.float32),
        mesh=plsc.VectorSubcoreMesh(**_MESH),
        compiler_params=_SC_PARAMS,
        scratch_types=[
            pltpu.VMEM((ck,), jnp.int32),
            pltpu.VMEM((ck,), jnp.int32),
            pltpu.VMEM((ck,), jnp.float32),
            pltpu.VMEM((784,), jnp.float32),
            pltpu.VMEM_SHARED((NPAD,), jnp.float32),
            pltpu.SemaphoreType.DMA,
            pltpu.SemaphoreType.DMA,
        ],
    )
    def k(ei_hbm, out_hbm, dst0_v, dst1_v, ones_v, zb_v, acc_sh, s0, s1):
        c = lax.axis_index("c")
        s = lax.axis_index("s")
        r0 = s * RPT
        dbuf = (dst0_v, dst1_v)
        sem = (s0, s1)

        def fill(i, carry):
            ones_v[pl.ds(i * 16, 16)] = jnp.ones((16,), jnp.float32)
            return carry

        lax.fori_loop(0, ck // 16, fill, 0)

        def zfill(i, carry):
            zb_v[pl.ds(i * 16, 16)] = jnp.zeros((16,), jnp.float32)
            return carry

        lax.fori_loop(0, 784 // 16, zfill, 0)

        def zcp(i, carry):
            pltpu.sync_copy(zb_v, acc_sh.at[pl.ds(r0 + i * 784, 784)])
            return carry

        lax.fori_loop(0, RPT // 784, zcp, 0)
        plsc.subcore_barrier()

        def idx(j, b):
            base = (c * NS + s) * ept + j * ck
            pltpu.sync_copy(ei_hbm.at[1, pl.ds(base, ck)], dbuf[b])

        def sstart(b):
            pltpu.async_copy(ones_v, acc_sh.at[dbuf[b]], sem[b], add=True)

        def swait(b):
            pltpu.make_async_copy(ones_v, acc_sh.at[dbuf[b]], sem[b]).wait()

        idx(0, 0)

        def body(j2, carry):
            e = 2 * j2
            idx(e + 1, 1)
            sstart(0)
            swait(0)

            @pl.when(e + 2 < nch)
            def _():
                idx(e + 2, 0)

            sstart(1)
            swait(1)
            return carry

        lax.fori_loop(0, nch // 2, body, 0)
        if nch % 2 == 1:
            sstart(0)
            swait(0)
        plsc.subcore_barrier()
        pltpu.sync_copy(acc_sh.at[pl.ds(r0, RPT)],
                        out_hbm.at[c, pl.ds(r0, RPT)])

    return k(ei)


def _sc_conv(y, ei, zrows, F, split_edges, ck):
    if split_edges:
        ept = E // (NC * NS)
        estride = E // NC
        off_core = False
    else:
        ept = E // NS
        estride = 0
        off_core = True
    nch = ept // ck

    @functools.partial(
        pl.kernel,
        out_type=jax.ShapeDtypeStruct((NC, NPAD, F), jnp.float32),
        mesh=plsc.VectorSubcoreMesh(**_MESH),
        compiler_params=_SC_PARAMS,
        scratch_types=[
            pltpu.VMEM((ck,), jnp.int32),
            pltpu.VMEM((ck,), jnp.int32),
            pltpu.VMEM((ck,), jnp.int32),
            pltpu.VMEM((ck,), jnp.int32),
            pltpu.VMEM((ck, F), jnp.float32),
            pltpu.VMEM((ck, F), jnp.float32),
            pltpu.VMEM_SHARED((NPAD, F), jnp.float32),
            pltpu.SemaphoreType.DMA,
            pltpu.SemaphoreType.DMA,
            pltpu.SemaphoreType.DMA,
            pltpu.SemaphoreType.DMA,
        ],
    )
    def k(y_hbm, ei_hbm, z_hbm, out_hbm,
          src0_v, src1_v, dst0_v, dst1_v, rows0_v, rows1_v, acc_sh,
          g0, g1, s0, s1):
        c = lax.axis_index("c")
        s = lax.axis_index("s")
        r0 = s * RPT
        sbuf = (src0_v, src1_v)
        dbuf = (dst0_v, dst1_v)
        rbuf = (rows0_v, rows1_v)
        gsem = (g0, g1)
        ssem = (s0, s1)

        if F == 16:
            def zfill(i, carry):
                rows0_v[i, :] = jnp.zeros((16,), jnp.float32)
                return carry

            lax.fori_loop(0, 784, zfill, 0)

            def zcp(i, carry):
                pltpu.sync_copy(rows0_v.at[pl.ds(0, 784)],
                                acc_sh.at[pl.ds(r0 + i * 784, 784)])
                return carry

            lax.fori_loop(0, RPT // 784, zcp, 0)
        else:
            pltpu.sync_copy(z_hbm.at[pl.ds(r0, RPT)],
                            acc_sh.at[pl.ds(r0, RPT)])
        plsc.subcore_barrier()

        def idx(j, b):
            base = s * ept + j * ck
            pltpu.sync_copy(ei_hbm.at[0, pl.ds(c * estride + base, ck)],
                            sbuf[b])
            pltpu.sync_copy(ei_hbm.at[1, pl.ds(c * estride + base, ck)],
                            dbuf[b])
            if off_core:
                off = jnp.full((16,), c * NPAD, jnp.int32)

                def addoff(i, carry):
                    sbuf[b][pl.ds(i * 16, 16)] = (
                        sbuf[b][pl.ds(i * 16, 16)] + off)
                    return carry

                lax.fori_loop(0, ck // 16, addoff, 0)

        def gstart(b):
            pltpu.async_copy(y_hbm.at[sbuf[b]], rbuf[b], gsem[b])

        def gwait(b):
            pltpu.make_async_copy(y_hbm.at[sbuf[b]], rbuf[b],
                                  gsem[b]).wait()

        def sstart(b):
            pltpu.async_copy(rbuf[b], acc_sh.at[dbuf[b]], ssem[b], add=True)

        def swait(b):
            pltpu.make_async_copy(rbuf[b], acc_sh.at[dbuf[b]],
                                  ssem[b]).wait()

        idx(0, 0)
        gstart(0)

        def body(j2, carry):
            e = 2 * j2
            idx(e + 1, 1)
            gstart(1)
            gwait(0)
            sstart(0)
            swait(0)

            @pl.when(e + 2 < nch)
            def _():
                idx(e + 2, 0)
                gstart(0)

            gwait(1)
            sstart(1)
            swait(1)
            return carry

        lax.fori_loop(0, nch // 2, body, 0)
        if nch % 2 == 1:
            gwait(0)
            sstart(0)
            swait(0)
        plsc.subcore_barrier()
        pltpu.sync_copy(acc_sh.at[pl.ds(r0, RPT)],
                        out_hbm.at[c, pl.ds(r0, RPT)])

    return k(y, ei, zrows)


def _tc_head(xp, degp, W1, b1, Wc1):
    def body(x_ref, degp_ref, W1_ref, b1_ref, Wc1_ref, y_ref, dinv_ref):
        deg = degp_ref[0, :] + degp_ref[1, :] + 1.0
        dinv = lax.rsqrt(deg)
        h = jnp.maximum(
            jnp.dot(x_ref[...], W1_ref[...],
                    preferred_element_type=jnp.float32) + b1_ref[...][None, :],
            0.0)
        xw = jnp.dot(h, Wc1_ref[...], preferred_element_type=jnp.float32)
        y = xw * dinv[:, None]
        y_ref[0, :, :] = y[:, :16]
        y_ref[1, :, :] = y[:, 16:]
        dinv_ref[0, 0, :] = dinv

    return pl.pallas_call(
        body,
        grid=(NB,),
        in_specs=[
            pl.BlockSpec((TB, 12), lambda i: (i, 0)),
            pl.BlockSpec((2, TB), lambda i: (0, i)),
            pl.BlockSpec((12, 32), lambda i: (0, 0)),
            pl.BlockSpec((32,), lambda i: (0,)),
            pl.BlockSpec((32, 32), lambda i: (0, 0)),
        ],
        out_specs=[
            pl.BlockSpec((2, TB, 16), lambda i: (0, i, 0)),
            pl.BlockSpec((1, 1, TB), lambda i: (i, 0, 0)),
        ],
        out_shape=[
            jax.ShapeDtypeStruct((2, NPAD, 16), jnp.float32),
            jax.ShapeDtypeStruct((NB, 1, TB), jnp.float32),
        ],
        compiler_params=_TC_PARAMS,
    )(xp, degp, W1, b1, Wc1)


def _tc_mid(agg1, y1, dinv, bc1, Wc3p):
    def body(agg_ref, y1_ref, dinv_ref, bc1_ref, Wc3_ref, y2_ref):
        dinv = dinv_ref[0, 0, :]
        lo = jnp.maximum(
            dinv[:, None] * (agg_ref[0] + y1_ref[0]) + bc1_ref[0:16][None, :],
            0.0)
        hi = jnp.maximum(
            dinv[:, None] * (agg_ref[1] + y1_ref[1]) + bc1_ref[16:32][None, :],
            0.0)
        xw = (jnp.dot(lo, Wc3_ref[0:16, :], preferred_element_type=jnp.float32)
              + jnp.dot(hi, Wc3_ref[16:32, :],
                        preferred_element_type=jnp.float32))
        y2_ref[...] = xw * dinv[:, None]

    return pl.pallas_call(
        body,
        grid=(NB,),
        in_specs=[
            pl.BlockSpec((2, TB, 16), lambda i: (0, i, 0)),
            pl.BlockSpec((2, TB, 16), lambda i: (0, i, 0)),
            pl.BlockSpec((1, 1, TB), lambda i: (i, 0, 0)),
            pl.BlockSpec((32,), lambda i: (0,)),
            pl.BlockSpec((32, 8), lambda i: (0, 0)),
        ],
        out_specs=pl.BlockSpec((TB, 8), lambda i: (i, 0)),
        out_shape=jax.ShapeDtypeStruct((NPAD, 8), jnp.float32),
        compiler_params=_TC_PARAMS,
    )(agg1, y1, dinv, bc1, Wc3p)


def _tc_tail(agg2, y2, dinv, bc3p, batchp):
    def body(agg_ref, y2_ref, dinv_ref, bc3_ref, batch_ref, out_ref):
        i = pl.program_id(0)
        dinv = dinv_ref[0, 0, :]
        h3 = (dinv[:, None] * (agg_ref[0] + agg_ref[1] + y2_ref[...])
              + bc3_ref[...][None, :])
        oh = (batch_ref[0, 0, :][:, None]
              == lax.broadcasted_iota(jnp.int32, (1, G), 1)).astype(jnp.float32)
        blk = lax.dot_general(oh, h3, (((0,), (0,)), ((), ())),
                              preferred_element_type=jnp.float32)

        @pl.when(i == 0)
        def _():
            out_ref[...] = jnp.zeros((G, 8), jnp.float32)

        out_ref[...] += blk

        @pl.when(i == NB - 1)
        def _():
            p = out_ref[...]
            col = lax.broadcasted_iota(jnp.int32, (G, 8), 1)
            pm = jnp.where(col < 3, p, -jnp.inf)
            mx = jnp.max(pm, axis=1, keepdims=True)
            lse = mx + jnp.log(
                jnp.sum(jnp.where(col < 3, jnp.exp(p - mx), 0.0),
                        axis=1, keepdims=True))
            out_ref[...] = p - lse

    return pl.pallas_call(
        body,
        grid=(NB,),
        in_specs=[
            pl.BlockSpec((2, TB, 8), lambda i: (0, i, 0)),
            pl.BlockSpec((TB, 8), lambda i: (i, 0)),
            pl.BlockSpec((1, 1, TB), lambda i: (i, 0, 0)),
            pl.BlockSpec((8,), lambda i: (0,)),
            pl.BlockSpec((1, 1, TB), lambda i: (i, 0, 0)),
        ],
        out_specs=pl.BlockSpec((G, 8), lambda i: (0, 0)),
        out_shape=jax.ShapeDtypeStruct((G, 8), jnp.float32),
        compiler_params=_TC_PARAMS,
    )(agg2, y2, dinv, bc3p, batchp)


def kernel(x, edge_index, batch, W1, b1, Wc1, bc1, Wc3, bc3):
    xp = jnp.pad(x, ((0, NPAD - N), (0, 0)))
    batchp = jnp.pad(batch, (0, NPAD - N),
                     constant_values=G).reshape(NB, 1, TB)
    Wc3p = jnp.pad(Wc3, ((0, 0), (0, 5)))
    bc3p = jnp.pad(bc3, (0, 5))
    z8 = jnp.zeros((NPAD, 8), jnp.float32)

    degp = _sc_deg(edge_index)
    y1, dinv = _tc_head(xp, degp, W1, b1, Wc1)
    agg1 = _sc_conv(y1.reshape(2 * NPAD, 16), edge_index, z8,
                    F=16, split_edges=False, ck=800)
    y2 = _tc_mid(agg1, y1, dinv, bc1, Wc3p)
    agg2 = _sc_conv(y2, edge_index, z8, F=8, split_edges=True, ck=1000)
    pooled = _tc_tail(agg2, y2, dinv, bc3p, batchp)
    return pooled[:, :3]
